# Initial kernel scaffold; baseline (speedup 1.0000x reference)
import jax
import jax.numpy as jnp
from jax.experimental import pallas as pl


def kernel(output, target):
    raise NotImplementedError("write your pallas kernel here")


# speedup vs baseline: 12.6802x; 12.6802x over previous
import functools
import math

import jax
import jax.numpy as jnp
from jax import lax
from jax.experimental import pallas as pl
from jax.experimental.pallas import tpu as pltpu
from jax.experimental.pallas import tpu_sc as plsc

_NB, _NA, _NC, _NH, _NW = 32, 5, 20, 19, 19
_S = _NH * _NW
_CH = 5 + _NC
_MAXO = 30
_SIL_FACTOR = 0.6 / 1.6
_OBJ = 5.0

_ANCHORS_PX = [42.3072, 55.4064, 102.168, 128.302, 161.788, 259.165,
               303.076, 154.897, 359.565, 320.227]
_AWS = [_ANCHORS_PX[2 * n] / 32.0 for n in range(_NA)]
_AHS = [_ANCHORS_PX[2 * n + 1] / 32.0 for n in range(_NA)]
_LN2 = 0.6931471805599453
_SQRT2 = 1.4142135623730951


def _tc_dense_body(tref, oref, out_ref):
    X = oref[0, :, 0, :]
    Y = oref[0, :, 1, :]
    W = oref[0, :, 2, :]
    H = oref[0, :, 3, :]
    C = oref[0, :, 4, :]
    cellidx = lax.broadcasted_iota(jnp.int32, (_NA, _S), 1)
    ii = (cellidx % _NW).astype(jnp.float32)
    jj = (cellidx // _NW).astype(jnp.float32)
    rowidx = lax.broadcasted_iota(jnp.int32, (_NA, 1), 0)
    awcol = jnp.full((_NA, 1), _AWS[0], jnp.float32)
    ahcol = jnp.full((_NA, 1), _AHS[0], jnp.float32)
    for n in range(1, _NA):
        awcol = jnp.where(rowidx == n, _AWS[n], awcol)
        ahcol = jnp.where(rowidx == n, _AHS[n], ahcol)
    px = 1.0 / (1.0 + jnp.exp(-X)) + ii
    py = 1.0 / (1.0 + jnp.exp(-Y)) + jj
    pw = jnp.exp(W) * awcol
    ph = jnp.exp(H) * ahcol
    parea = pw * ph
    xl = px - pw / 2.0
    xr = px + pw / 2.0
    yl = py - ph / 2.0
    yr = py + ph / 2.0
    conf = 1.0 / (1.0 + jnp.exp(-C))

    def body(t, carry):
        anyh, vc = carry
        raw = tref[0, 0, 48 + t]
        vc = jnp.logical_and(vc, raw != 0.0)
        gxs = raw * 19.0
        gys = tref[0, 0, 96 + t] * 19.0
        gws = tref[0, 0, 144 + t] * 19.0
        ghs = tref[0, 0, 192 + t] * 19.0
        hgw = gws / 2.0
        hgh = ghs / 2.0
        mx = jnp.minimum(xl, gxs - hgw)
        Mx = jnp.maximum(xr, gxs + hgw)
        my = jnp.minimum(yl, gys - hgh)
        My = jnp.maximum(yr, gys + hgh)
        uw = Mx - mx
        uh = My - my
        cw = (pw + gws) - uw
        ch_ = (ph + ghs) - uh
        cond = (cw > 0.0) & (ch_ > 0.0) & ((cw * ch_) > _SIL_FACTOR * (parea + gws * ghs))
        anyh = jnp.maximum(anyh, jnp.where(jnp.logical_and(cond, vc), 1.0, 0.0))
        return anyh, vc

    anyh0 = jnp.zeros((_NA, _S), jnp.float32)
    anyh, _ = lax.fori_loop(0, _MAXO, body, (anyh0, jnp.asarray(True)))
    cmb = 1.0 - anyh
    out_ref[0, 0, 0] = 0.5 * jnp.sum(cmb * conf * conf)


def _sc_log(v):
    bits = plsc.bitcast(v, jnp.int32)
    e = ((bits >> 23) & 0xFF) - 127
    mbits = (bits & 0x007FFFFF) | 0x3F800000
    mf = plsc.bitcast(mbits, jnp.float32)
    big = mf > _SQRT2
    mf = jnp.where(big, mf * 0.5, mf)
    e = e + big.astype(jnp.int32)
    r = (mf - 1.0) / (mf + 1.0)
    r2 = r * r
    p = 1.0 / 7.0 + r2 * (1.0 / 9.0)
    p = 1.0 / 5.0 + r2 * p
    p = 1.0 / 3.0 + r2 * p
    p = 1.0 + r2 * p
    return e.astype(jnp.float32) * _LN2 + 2.0 * r * p


def _sc_body(out3_hbm, tgt_hbm, out_hbm, obuf, trow, keys, orow):
    cid = lax.axis_index("c")
    sid = lax.axis_index("s")
    b = sid * 2 + cid
    pltpu.sync_copy(tgt_hbm.at[b], trow)
    pltpu.sync_copy(out3_hbm.at[b], obuf)

    acc = jnp.zeros((16,), jnp.float32)
    chunk_state = []
    for ci in range(2):
        toff = 16 * ci
        tv = jnp.arange(16, dtype=jnp.int32) + toff
        clsf = trow[0, pl.ds(toff, 16)]
        gxr = trow[1, pl.ds(toff, 16)]
        gyr = trow[2, pl.ds(toff, 16)]
        gwr = trow[3, pl.ds(toff, 16)]
        ghr = trow[4, pl.ds(toff, 16)]
        gx = gxr * 19.0
        gy = gyr * 19.0
        gw = gwr * 19.0
        gh = ghr * 19.0

        halfgw = gw / 2.0
        halfgh = gh / 2.0
        garea = gw * gh
        best_iou = jnp.zeros((16,), jnp.float32)
        bestn = jnp.zeros((16,), jnp.int32)
        awb = jnp.full((16,), _AWS[0], jnp.float32)
        ahb = jnp.full((16,), _AHS[0], jnp.float32)
        for n in range(_NA):
            aw_n, ah_n = _AWS[n], _AHS[n]
            mx = jnp.minimum(-aw_n / 2.0, -halfgw)
            Mx = jnp.maximum(aw_n / 2.0, halfgw)
            my = jnp.minimum(-ah_n / 2.0, -halfgh)
            My = jnp.maximum(ah_n / 2.0, halfgh)
            uw = Mx - mx
            uh = My - my
            cw = (aw_n + gw) - uw
            ch_ = (ah_n + gh) - uh
            carea = jnp.where((cw <= 0.0) | (ch_ <= 0.0), 0.0, cw * ch_)
            uarea = (aw_n * ah_n + garea) - carea
            iou = carea / uarea
            upd = iou > best_iou
            best_iou = jnp.where(upd, iou, best_iou)
            bestn = jnp.where(upd, n, bestn)
            awb = jnp.where(upd, aw_n, awb)
            ahb = jnp.where(upd, ah_n, ahb)

        gi = gx.astype(jnp.int32)
        gj = gy.astype(jnp.int32)
        gic = jnp.clip(gi, 0, _NW - 1)
        gjc = jnp.clip(gj, 0, _NH - 1)
        cell = gjc * _NW + gic
        key = bestn * _S + cell
        keys[pl.ds(toff, 16)] = key
        chunk_state.append((tv, clsf, gx, gy, gw, gh, halfgw, halfgh, garea,
                            bestn, awb, ahb, gi, gj, gic, gjc, cell, key))

    for ci in range(2):
        (tv, clsf, gx, gy, gw, gh, halfgw, halfgh, garea,
         bestn, awb, ahb, gi, gj, gic, gjc, cell, key) = chunk_state[ci]
        colbase = bestn * (_CH * _S) + cell
        xc = plsc.load_gather(obuf, [colbase])
        yc = plsc.load_gather(obuf, [colbase + _S])
        wc = plsc.load_gather(obuf, [colbase + 2 * _S])
        hc = plsc.load_gather(obuf, [colbase + 3 * _S])
        cc = plsc.load_gather(obuf, [colbase + 4 * _S])
        cvs = [plsc.load_gather(obuf, [colbase + (5 + k) * _S]) for k in range(_NC)]

        sxc = 1.0 / (1.0 + jnp.exp(-xc))
        syc = 1.0 / (1.0 + jnp.exp(-yc))
        scc = 1.0 / (1.0 + jnp.exp(-cc))
        pxg = sxc + gic.astype(jnp.float32)
        pyg = syc + gjc.astype(jnp.float32)
        pwg = jnp.exp(wc) * awb
        phg = jnp.exp(hc) * ahb
        parea_l = pwg * phg
        halfpw = pwg / 2.0
        halfph = phg / 2.0
        xl_l = pxg - halfpw
        xr_l = pxg + halfpw
        yl_l = pyg - halfph
        yr_l = pyg + halfph

        mx = jnp.minimum(gx - halfgw, xl_l)
        Mx = jnp.maximum(gx + halfgw, xr_l)
        my = jnp.minimum(gy - halfgh, yl_l)
        My = jnp.maximum(gy + halfgh, yr_l)
        uw = Mx - mx
        uh = My - my
        cw = (gw + pwg) - uw
        ch_ = (gh + phg) - uh
        carea = jnp.where((cw <= 0.0) | (ch_ <= 0.0), 0.0, cw * ch_)
        uarea = (garea + parea_l) - carea
        tconf = carea / uarea

        def body(t2, carry):
            anyh, win, validv, vc = carry
            raw = trow[1, pl.ds(t2, 16)][0]
            vc2 = jnp.logical_and(vc, raw != 0.0)
            gxs = raw * 19.0
            gys = trow[2, pl.ds(t2, 16)][0] * 19.0
            gws = trow[3, pl.ds(t2, 16)][0] * 19.0
            ghs = trow[4, pl.ds(t2, 16)][0] * 19.0
            hgw = gws * 0.5
            hgh = ghs * 0.5
            mx2 = jnp.minimum(xl_l, gxs - hgw)
            Mx2 = jnp.maximum(xr_l, gxs + hgw)
            my2 = jnp.minimum(yl_l, gys - hgh)
            My2 = jnp.maximum(yr_l, gys + hgh)
            uw2 = Mx2 - mx2
            uh2 = My2 - my2
            cw2 = (pwg + gws) - uw2
            ch2 = (phg + ghs) - uh2
            cond = (cw2 > 0.0) & (ch2 > 0.0) & ((cw2 * ch2) > _SIL_FACTOR * (parea_l + gws * ghs))
            anyh = anyh | (cond & vc2)
            keyt = keys[pl.ds(t2, 16)][0]
            over = (keyt == key) & (t2 > tv) & vc2
            win = win & jnp.logical_not(over)
            validv = validv | ((tv == t2) & vc2)
            return anyh, win, validv, vc2

        anyh, win, validv, _ = lax.fori_loop(
            0, _MAXO, body,
            (jnp.zeros((16,), jnp.bool_), jnp.ones((16,), jnp.bool_),
             jnp.zeros((16,), jnp.bool_), jnp.asarray(True)))
        win = win & validv
        cmb = jnp.where(anyh, 0.0, 1.0)

        txv = gx - gi.astype(jnp.float32)
        tyv = gy - gj.astype(jnp.float32)
        twv = _sc_log(gw / awb)
        thv = _sc_log(gh / ahb)
        dx = sxc - txv
        dy = syc - tyv
        dw = wc - twv
        dh = hc - thv
        coord = dx * dx + dy * dy + dw * dw + dh * dh
        dconf = scc - tconf
        confc = _OBJ * dconf * dconf - cmb * scc * scc

        m = cvs[0]
        for k in range(1, _NC):
            m = jnp.maximum(m, cvs[k])
        ssum = jnp.exp(cvs[0] - m)
        for k in range(1, _NC):
            ssum = ssum + jnp.exp(cvs[k] - m)
        lse = m + _sc_log(ssum)
        ci_idx = clsf.astype(jnp.int32)
        picked = jnp.zeros((16,), jnp.float32)
        for k in range(_NC):
            picked = jnp.where(ci_idx == k, cvs[k], picked)

        term = 0.5 * coord + 0.5 * confc + (lse - picked)
        acc = acc + jnp.where(win, term, 0.0)

    orow[...] = acc
    pltpu.sync_copy(orow, out_hbm.at[b])


@functools.cache
def _sc_call():
    return pl.kernel(
        _sc_body,
        out_type=jax.ShapeDtypeStruct((_NB, 16), jnp.float32),
        mesh=plsc.VectorSubcoreMesh(core_axis_name="c", subcore_axis_name="s"),
        compiler_params=pltpu.CompilerParams(needs_layout_passes=False),
        scratch_types=[
            pltpu.VMEM((_NA * _CH * _S,), jnp.float32),
            pltpu.VMEM((5, 48), jnp.float32),
            pltpu.VMEM((48,), jnp.int32),
            pltpu.VMEM((16,), jnp.float32),
        ],
    )

_TC_CALL = pl.pallas_call(
    _tc_dense_body,
    grid=(_NB,),
    in_specs=[
        pl.BlockSpec((1, 1, 240), lambda b: (b, 0, 0), memory_space=pltpu.SMEM),
        pl.BlockSpec((1, _NA, 8, _S), lambda b: (b, 0, 0, 0)),
    ],
    out_specs=pl.BlockSpec((1, 1, 1), lambda b: (b, 0, 0), memory_space=pltpu.SMEM),
    out_shape=jax.ShapeDtypeStruct((_NB, 1, 1), jnp.float32),
)


def kernel(output, target):
    out3 = output.reshape(_NB, _NA * _CH * _S)
    out4 = output.reshape(_NB, _NA, _CH, _S)
    t3 = jnp.pad(target.reshape(_NB, _MAXO, 5).transpose(0, 2, 1),
                 ((0, 0), (0, 0), (0, 18)))
    t240 = t3.reshape(_NB, 1, 240)
    dense = _TC_CALL(t240, out4)
    sparse = _sc_call()(out3, t3)
    return jnp.sum(dense) + jnp.sum(sparse)


# speedup vs baseline: 13.3222x; 1.0506x over previous
import functools
import math

import jax
import jax.numpy as jnp
from jax import lax
from jax.experimental import pallas as pl
from jax.experimental.pallas import tpu as pltpu
from jax.experimental.pallas import tpu_sc as plsc

_NB, _NA, _NC, _NH, _NW = 32, 5, 20, 19, 19
_S = _NH * _NW
_CH = 5 + _NC
_MAXO = 30
_SIL_FACTOR = 0.6 / 1.6
_OBJ = 5.0

_ANCHORS_PX = [42.3072, 55.4064, 102.168, 128.302, 161.788, 259.165,
               303.076, 154.897, 359.565, 320.227]
_AWS = [_ANCHORS_PX[2 * n] / 32.0 for n in range(_NA)]
_AHS = [_ANCHORS_PX[2 * n + 1] / 32.0 for n in range(_NA)]
_LN2 = 0.6931471805599453
_SQRT2 = 1.4142135623730951


def _tc_dense_body(tref, oref, out_ref):
    X = oref[0, :, 0, :]
    Y = oref[0, :, 1, :]
    W = oref[0, :, 2, :]
    H = oref[0, :, 3, :]
    C = oref[0, :, 4, :]
    cellidx = lax.broadcasted_iota(jnp.int32, (_NA, _S), 1)
    ii = (cellidx % _NW).astype(jnp.float32)
    jj = (cellidx // _NW).astype(jnp.float32)
    rowidx = lax.broadcasted_iota(jnp.int32, (_NA, 1), 0)
    awcol = jnp.full((_NA, 1), _AWS[0], jnp.float32)
    ahcol = jnp.full((_NA, 1), _AHS[0], jnp.float32)
    for n in range(1, _NA):
        awcol = jnp.where(rowidx == n, _AWS[n], awcol)
        ahcol = jnp.where(rowidx == n, _AHS[n], ahcol)
    px = 1.0 / (1.0 + jnp.exp(-X)) + ii
    py = 1.0 / (1.0 + jnp.exp(-Y)) + jj
    pw = jnp.exp(W) * awcol
    ph = jnp.exp(H) * ahcol
    parea = pw * ph
    xl = px - pw / 2.0
    xr = px + pw / 2.0
    yl = py - ph / 2.0
    yr = py + ph / 2.0
    conf = 1.0 / (1.0 + jnp.exp(-C))

    def body(t, carry):
        anyh, vc = carry
        raw = tref[0, 0, 48 + t]
        vc = jnp.logical_and(vc, raw != 0.0)
        gxs = raw * 19.0
        gys = tref[0, 0, 96 + t] * 19.0
        gws = tref[0, 0, 144 + t] * 19.0
        ghs = tref[0, 0, 192 + t] * 19.0
        hgw = gws / 2.0
        hgh = ghs / 2.0
        mx = jnp.minimum(xl, gxs - hgw)
        Mx = jnp.maximum(xr, gxs + hgw)
        my = jnp.minimum(yl, gys - hgh)
        My = jnp.maximum(yr, gys + hgh)
        uw = Mx - mx
        uh = My - my
        cw = (pw + gws) - uw
        ch_ = (ph + ghs) - uh
        cond = (cw > 0.0) & (ch_ > 0.0) & ((cw * ch_) > _SIL_FACTOR * (parea + gws * ghs))
        anyh = jnp.maximum(anyh, jnp.where(jnp.logical_and(cond, vc), 1.0, 0.0))
        return anyh, vc

    anyh0 = jnp.zeros((_NA, _S), jnp.float32)
    anyh, _ = lax.fori_loop(0, _MAXO, body, (anyh0, jnp.asarray(True)),
                            unroll=6)
    cmb = 1.0 - anyh
    out_ref[0, 0, 0] = 0.5 * jnp.sum(cmb * conf * conf)


def _sc_log(v):
    bits = plsc.bitcast(v, jnp.int32)
    e = ((bits >> 23) & 0xFF) - 127
    mbits = (bits & 0x007FFFFF) | 0x3F800000
    mf = plsc.bitcast(mbits, jnp.float32)
    big = mf > _SQRT2
    mf = jnp.where(big, mf * 0.5, mf)
    e = e + big.astype(jnp.int32)
    r = (mf - 1.0) / (mf + 1.0)
    r2 = r * r
    p = 1.0 / 7.0 + r2 * (1.0 / 9.0)
    p = 1.0 / 5.0 + r2 * p
    p = 1.0 / 3.0 + r2 * p
    p = 1.0 + r2 * p
    return e.astype(jnp.float32) * _LN2 + 2.0 * r * p


def _sc_body(out3_hbm, tgt_hbm, out_hbm, obuf, trow, keys, orow):
    cid = lax.axis_index("c")
    sid = lax.axis_index("s")
    b = sid * 2 + cid
    pltpu.sync_copy(tgt_hbm.at[b], trow)
    pltpu.sync_copy(out3_hbm.at[b], obuf)

    acc = jnp.zeros((16,), jnp.float32)
    chunk_state = []
    for ci in range(2):
        toff = 16 * ci
        tv = jnp.arange(16, dtype=jnp.int32) + toff
        clsf = trow[0, pl.ds(toff, 16)]
        gxr = trow[1, pl.ds(toff, 16)]
        gyr = trow[2, pl.ds(toff, 16)]
        gwr = trow[3, pl.ds(toff, 16)]
        ghr = trow[4, pl.ds(toff, 16)]
        gx = gxr * 19.0
        gy = gyr * 19.0
        gw = gwr * 19.0
        gh = ghr * 19.0

        halfgw = gw / 2.0
        halfgh = gh / 2.0
        garea = gw * gh
        best_iou = jnp.zeros((16,), jnp.float32)
        bestn = jnp.zeros((16,), jnp.int32)
        awb = jnp.full((16,), _AWS[0], jnp.float32)
        ahb = jnp.full((16,), _AHS[0], jnp.float32)
        for n in range(_NA):
            aw_n, ah_n = _AWS[n], _AHS[n]
            mx = jnp.minimum(-aw_n / 2.0, -halfgw)
            Mx = jnp.maximum(aw_n / 2.0, halfgw)
            my = jnp.minimum(-ah_n / 2.0, -halfgh)
            My = jnp.maximum(ah_n / 2.0, halfgh)
            uw = Mx - mx
            uh = My - my
            cw = (aw_n + gw) - uw
            ch_ = (ah_n + gh) - uh
            carea = jnp.where((cw <= 0.0) | (ch_ <= 0.0), 0.0, cw * ch_)
            uarea = (aw_n * ah_n + garea) - carea
            iou = carea / uarea
            upd = iou > best_iou
            best_iou = jnp.where(upd, iou, best_iou)
            bestn = jnp.where(upd, n, bestn)
            awb = jnp.where(upd, aw_n, awb)
            ahb = jnp.where(upd, ah_n, ahb)

        gi = gx.astype(jnp.int32)
        gj = gy.astype(jnp.int32)
        gic = jnp.clip(gi, 0, _NW - 1)
        gjc = jnp.clip(gj, 0, _NH - 1)
        cell = gjc * _NW + gic
        key = bestn * _S + cell
        keys[pl.ds(toff, 16)] = key
        chunk_state.append((tv, clsf, gx, gy, gw, gh, halfgw, halfgh, garea,
                            bestn, awb, ahb, gi, gj, gic, gjc, cell, key))

    for ci in range(2):
        (tv, clsf, gx, gy, gw, gh, halfgw, halfgh, garea,
         bestn, awb, ahb, gi, gj, gic, gjc, cell, key) = chunk_state[ci]
        colbase = bestn * (_CH * _S) + cell
        xc = plsc.load_gather(obuf, [colbase])
        yc = plsc.load_gather(obuf, [colbase + _S])
        wc = plsc.load_gather(obuf, [colbase + 2 * _S])
        hc = plsc.load_gather(obuf, [colbase + 3 * _S])
        cc = plsc.load_gather(obuf, [colbase + 4 * _S])
        cvs = [plsc.load_gather(obuf, [colbase + (5 + k) * _S]) for k in range(_NC)]

        sxc = 1.0 / (1.0 + jnp.exp(-xc))
        syc = 1.0 / (1.0 + jnp.exp(-yc))
        scc = 1.0 / (1.0 + jnp.exp(-cc))
        pxg = sxc + gic.astype(jnp.float32)
        pyg = syc + gjc.astype(jnp.float32)
        pwg = jnp.exp(wc) * awb
        phg = jnp.exp(hc) * ahb
        parea_l = pwg * phg
        halfpw = pwg / 2.0
        halfph = phg / 2.0
        xl_l = pxg - halfpw
        xr_l = pxg + halfpw
        yl_l = pyg - halfph
        yr_l = pyg + halfph

        mx = jnp.minimum(gx - halfgw, xl_l)
        Mx = jnp.maximum(gx + halfgw, xr_l)
        my = jnp.minimum(gy - halfgh, yl_l)
        My = jnp.maximum(gy + halfgh, yr_l)
        uw = Mx - mx
        uh = My - my
        cw = (gw + pwg) - uw
        ch_ = (gh + phg) - uh
        carea = jnp.where((cw <= 0.0) | (ch_ <= 0.0), 0.0, cw * ch_)
        uarea = (garea + parea_l) - carea
        tconf = carea / uarea

        def body(t2, carry):
            anyh, win, validv, vc = carry
            raw = trow[1, pl.ds(t2, 16)][0]
            vc2 = jnp.logical_and(vc, raw != 0.0)
            gxs = raw * 19.0
            gys = trow[2, pl.ds(t2, 16)][0] * 19.0
            gws = trow[3, pl.ds(t2, 16)][0] * 19.0
            ghs = trow[4, pl.ds(t2, 16)][0] * 19.0
            hgw = gws * 0.5
            hgh = ghs * 0.5
            mx2 = jnp.minimum(xl_l, gxs - hgw)
            Mx2 = jnp.maximum(xr_l, gxs + hgw)
            my2 = jnp.minimum(yl_l, gys - hgh)
            My2 = jnp.maximum(yr_l, gys + hgh)
            uw2 = Mx2 - mx2
            uh2 = My2 - my2
            cw2 = (pwg + gws) - uw2
            ch2 = (phg + ghs) - uh2
            cond = (cw2 > 0.0) & (ch2 > 0.0) & ((cw2 * ch2) > _SIL_FACTOR * (parea_l + gws * ghs))
            anyh = anyh | (cond & vc2)
            keyt = keys[pl.ds(t2, 16)][0]
            over = (keyt == key) & (t2 > tv) & vc2
            win = win & jnp.logical_not(over)
            validv = validv | ((tv == t2) & vc2)
            return anyh, win, validv, vc2

        anyh, win, validv, _ = lax.fori_loop(
            0, _MAXO, body,
            (jnp.zeros((16,), jnp.bool_), jnp.ones((16,), jnp.bool_),
             jnp.zeros((16,), jnp.bool_), jnp.asarray(True)))
        win = win & validv
        cmb = jnp.where(anyh, 0.0, 1.0)

        txv = gx - gi.astype(jnp.float32)
        tyv = gy - gj.astype(jnp.float32)
        twv = _sc_log(gw / awb)
        thv = _sc_log(gh / ahb)
        dx = sxc - txv
        dy = syc - tyv
        dw = wc - twv
        dh = hc - thv
        coord = dx * dx + dy * dy + dw * dw + dh * dh
        dconf = scc - tconf
        confc = _OBJ * dconf * dconf - cmb * scc * scc

        m = cvs[0]
        for k in range(1, _NC):
            m = jnp.maximum(m, cvs[k])
        ssum = jnp.exp(cvs[0] - m)
        for k in range(1, _NC):
            ssum = ssum + jnp.exp(cvs[k] - m)
        lse = m + _sc_log(ssum)
        ci_idx = clsf.astype(jnp.int32)
        picked = jnp.zeros((16,), jnp.float32)
        for k in range(_NC):
            picked = jnp.where(ci_idx == k, cvs[k], picked)

        term = 0.5 * coord + 0.5 * confc + (lse - picked)
        acc = acc + jnp.where(win, term, 0.0)

    orow[...] = acc
    pltpu.sync_copy(orow, out_hbm.at[b])


@functools.cache
def _sc_call():
    return pl.kernel(
        _sc_body,
        out_type=jax.ShapeDtypeStruct((_NB, 16), jnp.float32),
        mesh=plsc.VectorSubcoreMesh(core_axis_name="c", subcore_axis_name="s"),
        compiler_params=pltpu.CompilerParams(needs_layout_passes=False),
        scratch_types=[
            pltpu.VMEM((_NA * _CH * _S,), jnp.float32),
            pltpu.VMEM((5, 48), jnp.float32),
            pltpu.VMEM((48,), jnp.int32),
            pltpu.VMEM((16,), jnp.float32),
        ],
    )

_TC_CALL = pl.pallas_call(
    _tc_dense_body,
    grid=(_NB,),
    in_specs=[
        pl.BlockSpec((1, 1, 240), lambda b: (b, 0, 0), memory_space=pltpu.SMEM),
        pl.BlockSpec((1, _NA, 8, _S), lambda b: (b, 0, 0, 0)),
    ],
    out_specs=pl.BlockSpec((1, 1, 1), lambda b: (b, 0, 0), memory_space=pltpu.SMEM),
    out_shape=jax.ShapeDtypeStruct((_NB, 1, 1), jnp.float32),
)


def kernel(output, target):
    out3 = output.reshape(_NB, _NA * _CH * _S)
    out4 = out3.reshape(_NB, _NA, _CH, _S)
    t3 = jnp.pad(target.reshape(_NB, _MAXO, 5).transpose(0, 2, 1),
                 ((0, 0), (0, 0), (0, 18)))
    t240 = t3.reshape(_NB, 1, 240)
    dense = _TC_CALL(t240, out4)
    sparse = _sc_call()(out3, t3)
    return jnp.sum(dense) + jnp.sum(sparse)


# speedup vs baseline: 14.0596x; 1.0554x over previous
import functools
import math

import jax
import jax.numpy as jnp
from jax import lax
from jax.experimental import pallas as pl
from jax.experimental.pallas import tpu as pltpu
from jax.experimental.pallas import tpu_sc as plsc

_NB, _NA, _NC, _NH, _NW = 32, 5, 20, 19, 19
_S = _NH * _NW
_CH = 5 + _NC
_MAXO = 30
_SIL_FACTOR = 0.6 / 1.6
_OBJ = 5.0

_ANCHORS_PX = [42.3072, 55.4064, 102.168, 128.302, 161.788, 259.165,
               303.076, 154.897, 359.565, 320.227]
_AWS = [_ANCHORS_PX[2 * n] / 32.0 for n in range(_NA)]
_AHS = [_ANCHORS_PX[2 * n + 1] / 32.0 for n in range(_NA)]
_LN2 = 0.6931471805599453
_SQRT2 = 1.4142135623730951


def _tc_dense_body(tref, oref, out_ref):
    X = oref[0, 0:5, :]
    Y = oref[0, 5:10, :]
    W = oref[0, 10:15, :]
    H = oref[0, 15:20, :]
    C = oref[0, 20:25, :]
    cellidx = lax.broadcasted_iota(jnp.int32, (_NA, _S), 1)
    ii = (cellidx % _NW).astype(jnp.float32)
    jj = (cellidx // _NW).astype(jnp.float32)
    rowidx = lax.broadcasted_iota(jnp.int32, (_NA, 1), 0)
    awcol = jnp.full((_NA, 1), _AWS[0], jnp.float32)
    ahcol = jnp.full((_NA, 1), _AHS[0], jnp.float32)
    for n in range(1, _NA):
        awcol = jnp.where(rowidx == n, _AWS[n], awcol)
        ahcol = jnp.where(rowidx == n, _AHS[n], ahcol)
    px = 1.0 / (1.0 + jnp.exp(-X)) + ii
    py = 1.0 / (1.0 + jnp.exp(-Y)) + jj
    pw = jnp.exp(W) * awcol
    ph = jnp.exp(H) * ahcol
    parea = pw * ph
    xl = px - pw / 2.0
    xr = px + pw / 2.0
    yl = py - ph / 2.0
    yr = py + ph / 2.0
    conf = 1.0 / (1.0 + jnp.exp(-C))

    def body(t, carry):
        anyh, vc = carry
        raw = tref[0, 0, 48 + t]
        vc = jnp.logical_and(vc, raw != 0.0)
        gxs = raw * 19.0
        gys = tref[0, 0, 96 + t] * 19.0
        gws = tref[0, 0, 144 + t] * 19.0
        ghs = tref[0, 0, 192 + t] * 19.0
        hgw = gws / 2.0
        hgh = ghs / 2.0
        mx = jnp.minimum(xl, gxs - hgw)
        Mx = jnp.maximum(xr, gxs + hgw)
        my = jnp.minimum(yl, gys - hgh)
        My = jnp.maximum(yr, gys + hgh)
        uw = Mx - mx
        uh = My - my
        cw = (pw + gws) - uw
        ch_ = (ph + ghs) - uh
        cond = (cw > 0.0) & (ch_ > 0.0) & ((cw * ch_) > _SIL_FACTOR * (parea + gws * ghs))
        anyh = jnp.maximum(anyh, jnp.where(jnp.logical_and(cond, vc), 1.0, 0.0))
        return anyh, vc

    anyh0 = jnp.zeros((_NA, _S), jnp.float32)
    anyh, _ = lax.fori_loop(0, _MAXO, body, (anyh0, jnp.asarray(True)),
                            unroll=6)
    cmb = 1.0 - anyh
    out_ref[0, 0, 0] = 0.5 * jnp.sum(cmb * conf * conf)


def _sc_log(v):
    bits = plsc.bitcast(v, jnp.int32)
    e = ((bits >> 23) & 0xFF) - 127
    mbits = (bits & 0x007FFFFF) | 0x3F800000
    mf = plsc.bitcast(mbits, jnp.float32)
    big = mf > _SQRT2
    mf = jnp.where(big, mf * 0.5, mf)
    e = e + big.astype(jnp.int32)
    r = (mf - 1.0) / (mf + 1.0)
    r2 = r * r
    p = 1.0 / 7.0 + r2 * (1.0 / 9.0)
    p = 1.0 / 5.0 + r2 * p
    p = 1.0 / 3.0 + r2 * p
    p = 1.0 + r2 * p
    return e.astype(jnp.float32) * _LN2 + 2.0 * r * p


def _sc_body(out3_hbm, tgt_hbm, out_hbm, obuf, trow, keys, orow):
    cid = lax.axis_index("c")
    sid = lax.axis_index("s")
    b = sid * 2 + cid
    pltpu.sync_copy(tgt_hbm.at[b], trow)
    pltpu.sync_copy(out3_hbm.at[b], obuf)

    acc = jnp.zeros((16,), jnp.float32)
    chunk_state = []
    for ci in range(2):
        toff = 16 * ci
        tv = jnp.arange(16, dtype=jnp.int32) + toff
        clsf = trow[0, pl.ds(toff, 16)]
        gxr = trow[1, pl.ds(toff, 16)]
        gyr = trow[2, pl.ds(toff, 16)]
        gwr = trow[3, pl.ds(toff, 16)]
        ghr = trow[4, pl.ds(toff, 16)]
        gx = gxr * 19.0
        gy = gyr * 19.0
        gw = gwr * 19.0
        gh = ghr * 19.0

        halfgw = gw / 2.0
        halfgh = gh / 2.0
        garea = gw * gh
        best_iou = jnp.zeros((16,), jnp.float32)
        bestn = jnp.zeros((16,), jnp.int32)
        awb = jnp.full((16,), _AWS[0], jnp.float32)
        ahb = jnp.full((16,), _AHS[0], jnp.float32)
        for n in range(_NA):
            aw_n, ah_n = _AWS[n], _AHS[n]
            mx = jnp.minimum(-aw_n / 2.0, -halfgw)
            Mx = jnp.maximum(aw_n / 2.0, halfgw)
            my = jnp.minimum(-ah_n / 2.0, -halfgh)
            My = jnp.maximum(ah_n / 2.0, halfgh)
            uw = Mx - mx
            uh = My - my
            cw = (aw_n + gw) - uw
            ch_ = (ah_n + gh) - uh
            carea = jnp.where((cw <= 0.0) | (ch_ <= 0.0), 0.0, cw * ch_)
            uarea = (aw_n * ah_n + garea) - carea
            iou = carea / uarea
            upd = iou > best_iou
            best_iou = jnp.where(upd, iou, best_iou)
            bestn = jnp.where(upd, n, bestn)
            awb = jnp.where(upd, aw_n, awb)
            ahb = jnp.where(upd, ah_n, ahb)

        gi = gx.astype(jnp.int32)
        gj = gy.astype(jnp.int32)
        gic = jnp.clip(gi, 0, _NW - 1)
        gjc = jnp.clip(gj, 0, _NH - 1)
        cell = gjc * _NW + gic
        key = bestn * _S + cell
        keys[pl.ds(toff, 16)] = key
        chunk_state.append((tv, clsf, gx, gy, gw, gh, halfgw, halfgh, garea,
                            bestn, awb, ahb, gi, gj, gic, gjc, cell, key))

    for ci in range(2):
        (tv, clsf, gx, gy, gw, gh, halfgw, halfgh, garea,
         bestn, awb, ahb, gi, gj, gic, gjc, cell, key) = chunk_state[ci]
        colbase = bestn * (_CH * _S) + cell
        xc = plsc.load_gather(obuf, [colbase])
        yc = plsc.load_gather(obuf, [colbase + _S])
        wc = plsc.load_gather(obuf, [colbase + 2 * _S])
        hc = plsc.load_gather(obuf, [colbase + 3 * _S])
        cc = plsc.load_gather(obuf, [colbase + 4 * _S])
        cvs = [plsc.load_gather(obuf, [colbase + (5 + k) * _S]) for k in range(_NC)]

        sxc = 1.0 / (1.0 + jnp.exp(-xc))
        syc = 1.0 / (1.0 + jnp.exp(-yc))
        scc = 1.0 / (1.0 + jnp.exp(-cc))
        pxg = sxc + gic.astype(jnp.float32)
        pyg = syc + gjc.astype(jnp.float32)
        pwg = jnp.exp(wc) * awb
        phg = jnp.exp(hc) * ahb
        parea_l = pwg * phg
        halfpw = pwg / 2.0
        halfph = phg / 2.0
        xl_l = pxg - halfpw
        xr_l = pxg + halfpw
        yl_l = pyg - halfph
        yr_l = pyg + halfph

        mx = jnp.minimum(gx - halfgw, xl_l)
        Mx = jnp.maximum(gx + halfgw, xr_l)
        my = jnp.minimum(gy - halfgh, yl_l)
        My = jnp.maximum(gy + halfgh, yr_l)
        uw = Mx - mx
        uh = My - my
        cw = (gw + pwg) - uw
        ch_ = (gh + phg) - uh
        carea = jnp.where((cw <= 0.0) | (ch_ <= 0.0), 0.0, cw * ch_)
        uarea = (garea + parea_l) - carea
        tconf = carea / uarea

        def body(t2, carry):
            anyh, win, validv, vc = carry
            raw = trow[1, pl.ds(t2, 16)][0]
            vc2 = jnp.logical_and(vc, raw != 0.0)
            gxs = raw * 19.0
            gys = trow[2, pl.ds(t2, 16)][0] * 19.0
            gws = trow[3, pl.ds(t2, 16)][0] * 19.0
            ghs = trow[4, pl.ds(t2, 16)][0] * 19.0
            hgw = gws * 0.5
            hgh = ghs * 0.5
            mx2 = jnp.minimum(xl_l, gxs - hgw)
            Mx2 = jnp.maximum(xr_l, gxs + hgw)
            my2 = jnp.minimum(yl_l, gys - hgh)
            My2 = jnp.maximum(yr_l, gys + hgh)
            uw2 = Mx2 - mx2
            uh2 = My2 - my2
            cw2 = (pwg + gws) - uw2
            ch2 = (phg + ghs) - uh2
            cond = (cw2 > 0.0) & (ch2 > 0.0) & ((cw2 * ch2) > _SIL_FACTOR * (parea_l + gws * ghs))
            anyh = anyh | (cond & vc2)
            keyt = keys[pl.ds(t2, 16)][0]
            over = (keyt == key) & (t2 > tv) & vc2
            win = win & jnp.logical_not(over)
            validv = validv | ((tv == t2) & vc2)
            return anyh, win, validv, vc2

        anyh, win, validv, _ = lax.fori_loop(
            0, _MAXO, body,
            (jnp.zeros((16,), jnp.bool_), jnp.ones((16,), jnp.bool_),
             jnp.zeros((16,), jnp.bool_), jnp.asarray(True)))
        win = win & validv
        cmb = jnp.where(anyh, 0.0, 1.0)

        txv = gx - gi.astype(jnp.float32)
        tyv = gy - gj.astype(jnp.float32)
        twv = _sc_log(gw / awb)
        thv = _sc_log(gh / ahb)
        dx = sxc - txv
        dy = syc - tyv
        dw = wc - twv
        dh = hc - thv
        coord = dx * dx + dy * dy + dw * dw + dh * dh
        dconf = scc - tconf
        confc = _OBJ * dconf * dconf - cmb * scc * scc

        m = cvs[0]
        for k in range(1, _NC):
            m = jnp.maximum(m, cvs[k])
        ssum = jnp.exp(cvs[0] - m)
        for k in range(1, _NC):
            ssum = ssum + jnp.exp(cvs[k] - m)
        lse = m + _sc_log(ssum)
        ci_idx = clsf.astype(jnp.int32)
        picked = jnp.zeros((16,), jnp.float32)
        for k in range(_NC):
            picked = jnp.where(ci_idx == k, cvs[k], picked)

        term = 0.5 * coord + 0.5 * confc + (lse - picked)
        acc = acc + jnp.where(win, term, 0.0)

    orow[...] = acc
    pltpu.sync_copy(orow, out_hbm.at[b])


@functools.cache
def _sc_call():
    return pl.kernel(
        _sc_body,
        out_type=jax.ShapeDtypeStruct((_NB, 16), jnp.float32),
        mesh=plsc.VectorSubcoreMesh(core_axis_name="c", subcore_axis_name="s"),
        compiler_params=pltpu.CompilerParams(needs_layout_passes=False),
        scratch_types=[
            pltpu.VMEM((_NA * _CH * _S,), jnp.float32),
            pltpu.VMEM((5, 48), jnp.float32),
            pltpu.VMEM((48,), jnp.int32),
            pltpu.VMEM((16,), jnp.float32),
        ],
    )

_TC_CALL = pl.pallas_call(
    _tc_dense_body,
    grid=(_NB,),
    in_specs=[
        pl.BlockSpec((1, 1, 240), lambda b: (b, 0, 0), memory_space=pltpu.SMEM),
        pl.BlockSpec((1, 25, _S), lambda b: (b, 0, 0)),
    ],
    out_specs=pl.BlockSpec((1, 1, 1), lambda b: (b, 0, 0), memory_space=pltpu.SMEM),
    out_shape=jax.ShapeDtypeStruct((_NB, 1, 1), jnp.float32),
)


_BOX_ROWS = tuple(a * _CH + c for c in range(5) for a in range(_NA))


def kernel(output, target):
    out3 = output.reshape(_NB, _NA * _CH * _S)
    box = jnp.take(output, jnp.asarray(_BOX_ROWS, jnp.int32),
                   axis=1).reshape(_NB, 25, _S)
    t3 = jnp.pad(target.reshape(_NB, _MAXO, 5).transpose(0, 2, 1),
                 ((0, 0), (0, 0), (0, 18)))
    t240 = t3.reshape(_NB, 1, 240)
    dense = _TC_CALL(t240, box)
    sparse = _sc_call()(out3, t3)
    return jnp.sum(dense) + jnp.sum(sparse)
